# Initial kernel scaffold; baseline (speedup 1.0000x reference)
import jax
import jax.numpy as jnp
from jax.experimental import pallas as pl


def kernel(i, j, table_i, table_j):
    raise NotImplementedError("write your pallas kernel here")


# speedup vs baseline: 1.0250x; 1.0250x over previous
import functools

import jax
import jax.numpy as jnp
from jax import lax
from jax.experimental import pallas as pl
from jax.experimental.pallas import tpu as pltpu
from jax.experimental.pallas import tpu_sc as plsc

B, H, W = 4, 224, 224
C = 128
NROW = 224
L = 16
NWB = W // L


def _body(i_hbm, j_hbm, ti_hbm, tj_hbm, out_hbm,
          ti_v, tj_v, idx_i, idx_j, outbuf):
    info = plsc.get_sparse_core_info()
    nc, ns = info.num_cores, info.num_subcores
    nw = nc * ns
    pairs_per_w = (B * H) // nw

    wid = lax.axis_index("s") * nc + lax.axis_index("c")

    pltpu.sync_copy(ti_hbm, ti_v)
    pltpu.sync_copy(tj_hbm, tj_v)

    def do_pair(p, _):
        pair = wid * pairs_per_w + p
        b = pair // H
        h = pair % H
        pltpu.sync_copy(i_hbm.at[b, h, :], idx_i)
        pltpu.sync_copy(j_hbm.at[b, h, :], idx_j)

        for wb in range(NWB):
            iv = idx_i[pl.ds(wb * L, L)]
            jv = idx_j[pl.ds(wb * L, L)]

            iv_base = iv * C
            jv_base = jv * C

            def do_chan(c, _):
                cv = jnp.full((L,), c, jnp.int32)
                vi = plsc.load_gather(ti_v, [iv_base + cv])
                outbuf[c, pl.ds(wb * L, L)] = vi
                vj = plsc.load_gather(tj_v, [jv_base + cv])
                outbuf[C + c, pl.ds(wb * L, L)] = vj
                return 0

            lax.fori_loop(0, C, do_chan, 0, unroll=4)

        pltpu.sync_copy(outbuf, out_hbm.at[b, :, h, :])
        return 0

    lax.fori_loop(0, pairs_per_w, do_pair, 0)


@jax.jit
def _position_embedding_sc(i, j, table_i, table_j):
    mesh = plsc.VectorSubcoreMesh(core_axis_name="c", subcore_axis_name="s")
    fn = pl.kernel(
        _body,
        out_type=jax.ShapeDtypeStruct((B, 2 * C, H, W), jnp.float32),
        mesh=mesh,
        scratch_types=[
            pltpu.VMEM((NROW * C,), jnp.float32),
            pltpu.VMEM((NROW * C,), jnp.float32),
            pltpu.VMEM((W,), jnp.int32),
            pltpu.VMEM((W,), jnp.int32),
            pltpu.VMEM((2 * C, W), jnp.float32),
        ],
        compiler_params=pltpu.CompilerParams(needs_layout_passes=False),
    )
    return fn(i, j, table_i.reshape(-1), table_j.reshape(-1))


def kernel(i, j, table_i, table_j):
    return _position_embedding_sc(i, j, table_i, table_j)


# speedup vs baseline: 1.5742x; 1.5358x over previous
import functools

import jax
import jax.numpy as jnp
from jax import lax
from jax.experimental import pallas as pl
from jax.experimental.pallas import tpu as pltpu
from jax.experimental.pallas import tpu_sc as plsc

B, H, W = 4, 224, 224
C = 128
NROW = 224
L = 16
NWB = W // L


def _body(i_hbm, j_hbm, ti_hbm, tj_hbm, out_hbm,
          ti_v, tj_v, idx_i, idx_j, outbuf):
    info = plsc.get_sparse_core_info()
    nc, ns = info.num_cores, info.num_subcores
    nw = nc * ns
    pairs_per_w = (B * H) // nw

    wid = lax.axis_index("s") * nc + lax.axis_index("c")

    pltpu.sync_copy(ti_hbm, ti_v)
    pltpu.sync_copy(tj_hbm, tj_v)

    def do_pair(p, _):
        pair = wid * pairs_per_w + p
        b = pair // H
        h = pair % H
        pltpu.sync_copy(i_hbm.at[b, h, :], idx_i)
        pltpu.sync_copy(j_hbm.at[b, h, :], idx_j)

        @plsc.parallel_loop(0, NWB, 1, unroll=1)
        def wb_loop(wb):
            off = wb * L
            iv_base = idx_i[pl.ds(off, L)] * C
            jv_base = idx_j[pl.ds(off, L)] * C
            for c in range(C):
                vi = plsc.load_gather(ti_v, [iv_base + c])
                outbuf[c, pl.ds(off, L)] = vi
                vj = plsc.load_gather(tj_v, [jv_base + c])
                outbuf[C + c, pl.ds(off, L)] = vj

        pltpu.sync_copy(outbuf, out_hbm.at[b, :, h, :])
        return 0

    lax.fori_loop(0, pairs_per_w, do_pair, 0)


@jax.jit
def _position_embedding_sc(i, j, table_i, table_j):
    mesh = plsc.VectorSubcoreMesh(core_axis_name="c", subcore_axis_name="s")
    fn = pl.kernel(
        _body,
        out_type=jax.ShapeDtypeStruct((B, 2 * C, H, W), jnp.float32),
        mesh=mesh,
        scratch_types=[
            pltpu.VMEM((NROW * C,), jnp.float32),
            pltpu.VMEM((NROW * C,), jnp.float32),
            pltpu.VMEM((W,), jnp.int32),
            pltpu.VMEM((W,), jnp.int32),
            pltpu.VMEM((2 * C, W), jnp.float32),
        ],
        compiler_params=pltpu.CompilerParams(needs_layout_passes=False),
    )
    return fn(i, j, table_i.reshape(-1), table_j.reshape(-1))


def kernel(i, j, table_i, table_j):
    return _position_embedding_sc(i, j, table_i, table_j)
